# baseline (speedup 1.0000x reference)
import functools
import jax
import jax.numpy as jnp
from jax.experimental import pallas as pl
from jax.experimental.pallas import tpu as pltpu


def _node_mlp_kernel(x_ref, w1a_ref, w1b_ref, b1_ref, a_ref, b_ref):
    x = x_ref[...]
    a_ref[...] = (
        jnp.dot(x, w1a_ref[...], preferred_element_type=jnp.float32)
        + b1_ref[...]
    )
    b_ref[...] = jnp.dot(x, w1b_ref[...], preferred_element_type=jnp.float32)


def _edge_kernel(
    src_ref, dst_ref, a_ref, b_ref, w2_ref, b2_ref, w3_ref, b3_ref,
    out_ref, msg_ref,
):
    i = pl.program_id(0)
    eb = msg_ref.shape[0]
    n = out_ref.shape[0]

    @pl.when(i == 0)
    def _init():
        out_ref[...] = jnp.full(out_ref.shape, -jnp.inf, jnp.float32)

    def gather_body(e, _):
        s = src_ref[0, 0, e]
        d = dst_ref[0, 0, e]
        msg_ref[pl.ds(e, 1), :] = (
            a_ref[pl.ds(d, 1), :] + b_ref[pl.ds(s, 1), :]
        )
        return _

    jax.lax.fori_loop(0, eb, gather_body, 0)

    h = jnp.maximum(msg_ref[...], 0.0)
    h = jnp.maximum(
        jnp.dot(h, w2_ref[...], preferred_element_type=jnp.float32)
        + b2_ref[...],
        0.0,
    )
    h = jnp.dot(h, w3_ref[...], preferred_element_type=jnp.float32) + b3_ref[...]
    msg_ref[...] = h

    def scatter_body(e, _):
        d = dst_ref[0, 0, e]
        out_ref[pl.ds(d, 1), :] = jnp.maximum(
            out_ref[pl.ds(d, 1), :], msg_ref[pl.ds(e, 1), :]
        )
        return _

    jax.lax.fori_loop(0, eb, scatter_body, 0)

    @pl.when(i == pl.num_programs(0) - 1)
    def _fixup():
        o = out_ref[...]
        out_ref[...] = jnp.where(jnp.isneginf(o), 0.0, o)


def kernel(x, edge_index, W1, b1, W2, b2, W3, b3):
    n, d = x.shape
    e = edge_index.shape[1]
    eb = 1280
    nblk = e // eb
    assert nblk * eb == e

    w1a = W1[:d] - W1[d:]
    w1b = W1[d:]
    src = edge_index[0].astype(jnp.int32).reshape(nblk, 1, eb)
    dst = edge_index[1].astype(jnp.int32).reshape(nblk, 1, eb)

    a, b = pl.pallas_call(
        _node_mlp_kernel,
        grid=(5,),
        in_specs=[
            pl.BlockSpec((n // 5, d), lambda i: (i, 0)),
            pl.BlockSpec((d, d), lambda i: (0, 0)),
            pl.BlockSpec((d, d), lambda i: (0, 0)),
            pl.BlockSpec((1, d), lambda i: (0, 0)),
        ],
        out_specs=[
            pl.BlockSpec((n // 5, d), lambda i: (i, 0)),
            pl.BlockSpec((n // 5, d), lambda i: (i, 0)),
        ],
        out_shape=[
            jax.ShapeDtypeStruct((n, d), jnp.float32),
            jax.ShapeDtypeStruct((n, d), jnp.float32),
        ],
    )(x, w1a, w1b, b1.reshape(1, d))

    out = pl.pallas_call(
        _edge_kernel,
        grid=(nblk,),
        in_specs=[
            pl.BlockSpec((1, 1, eb), lambda i: (i, 0, 0), memory_space=pltpu.SMEM),
            pl.BlockSpec((1, 1, eb), lambda i: (i, 0, 0), memory_space=pltpu.SMEM),
            pl.BlockSpec((n, d), lambda i: (0, 0)),
            pl.BlockSpec((n, d), lambda i: (0, 0)),
            pl.BlockSpec((d, d), lambda i: (0, 0)),
            pl.BlockSpec((1, d), lambda i: (0, 0)),
            pl.BlockSpec((d, d), lambda i: (0, 0)),
            pl.BlockSpec((1, d), lambda i: (0, 0)),
        ],
        out_specs=pl.BlockSpec((n, d), lambda i: (0, 0)),
        out_shape=jax.ShapeDtypeStruct((n, d), jnp.float32),
        scratch_shapes=[pltpu.VMEM((eb, d), jnp.float32)],
    )(src, dst, a, b, W2, b2.reshape(1, d), W3, b3.reshape(1, d))

    return out


# speedup vs baseline: 2.1843x; 2.1843x over previous
import functools
import jax
import jax.numpy as jnp
from jax import lax
from jax.experimental import pallas as pl
from jax.experimental.pallas import tpu as pltpu
import jax.experimental.pallas.tpu_sc as plsc

N = 10000
E = 320000
D = 128
NW = 32
EPW = E // NW
RPW = 320
SAC = 320
ACC_ROWS = 328
CAP = EPW
MAGIC = 52429
EBITS = 19
BATCH = 80
NBATCH = EPW // BATCH

_mesh = plsc.VectorSubcoreMesh(core_axis_name="c", subcore_axis_name="s")


def _node_mlp_kernel(x_ref, w1a_ref, w1b_ref, b1_ref, a_ref, b_ref):
    x = x_ref[...]
    a_ref[...] = (
        jnp.dot(x, w1a_ref[...], preferred_element_type=jnp.float32)
        + b1_ref[...]
    )
    b_ref[...] = jnp.dot(x, w1b_ref[...], preferred_element_type=jnp.float32)


def _edge_mlp_kernel(msg_ref, w2_ref, b2_ref, w3_ref, b3_ref, out_ref):
    h = jnp.maximum(msg_ref[...], 0.0)
    h = jnp.maximum(
        jnp.dot(h, w2_ref[...], preferred_element_type=jnp.float32)
        + b2_ref[...],
        0.0,
    )
    out_ref[...] = (
        jnp.dot(h, w3_ref[...], preferred_element_type=jnp.float32)
        + b3_ref[...]
    )


def _gather_bin_kernel(a_h, b_h, src_h, dst_h, msg_h, lists_h, counts_h,
                       srcv, dstv, bufa, bufb, pk, ps, sema, semb, semc):
    c = lax.axis_index("c")
    s = lax.axis_index("s")
    wid = s * 2 + c
    lane = lax.iota(jnp.int32, 16)
    ebase = wid * EPW

    def batch_body(t, carry):
        cnt_lo, cnt_hi = carry
        base = ebase + t * BATCH
        pltpu.sync_copy(dst_h.at[pl.ds(base, BATCH)], dstv)
        pltpu.sync_copy(src_h.at[pl.ds(base, BATCH)], srcv)
        cpa = pltpu.async_copy(a_h.at[dstv], bufa, sema)
        cpb = pltpu.async_copy(b_h.at[srcv], bufb, semb)
        cpa.wait()
        cpb.wait()

        def row_body(r, _):
            for cc in range(8):
                sl = pl.ds(cc * 16, 16)
                bufa[r, sl] = bufa[r, sl] + bufb[r, sl]
            return 0

        lax.fori_loop(0, BATCH, row_body, 0)
        pltpu.sync_copy(bufa, msg_h.at[pl.ds(base, BATCH)])

        for g in range(BATCH // 16):
            d16 = dstv[pl.ds(g * 16, 16)]
            bkt = (d16 * MAGIC) >> 24
            loc = d16 - bkt * RPW
            eid = (base + g * 16) + lane
            packed = loc * (1 << EBITS) + eid
            rank = jnp.zeros((16,), jnp.int32)
            hist_lo = jnp.zeros((16,), jnp.int32)
            hist_hi = jnp.zeros((16,), jnp.int32)
            for kk in range(16):
                skv = bkt.at[jnp.full((16,), kk, jnp.int32)].get(
                    mode="promise_in_bounds")
                eq = bkt == skv
                rank = rank + jnp.where(eq & (lane > kk), 1, 0)
                hist_lo = hist_lo + jnp.where(lane == skv, 1, 0)
                hist_hi = hist_hi + jnp.where((lane + 16) == skv, 1, 0)
            g_lo = cnt_lo.at[bkt & 15].get(mode="promise_in_bounds")
            g_hi = cnt_hi.at[bkt & 15].get(mode="promise_in_bounds")
            cnt16 = jnp.where(bkt < 16, g_lo, g_hi)
            pos = bkt * (NW * CAP) + wid * CAP + cnt16 + rank
            pk[...] = packed
            ps[...] = pos
            pltpu.async_copy(pk, lists_h.at[ps], semc).wait()
            cnt_lo = cnt_lo + hist_lo
            cnt_hi = cnt_hi + hist_hi
        return cnt_lo, cnt_hi

    z16 = jnp.zeros((16,), jnp.int32)
    cnt_lo, cnt_hi = lax.fori_loop(0, NBATCH, batch_body, (z16, z16))

    pk[...] = cnt_lo
    ps[...] = lane * NW + wid
    pltpu.async_copy(pk, counts_h.at[ps], semc).wait()
    pk[...] = cnt_hi
    ps[...] = (lane + 16) * NW + wid
    pltpu.async_copy(pk, counts_h.at[ps], semc).wait()


def _segmax_kernel(h3_h, lists_h, counts_h, out_h,
                   cntv, pkv, idxv, rowbuf, acc, semg):
    c = lax.axis_index("c")
    s = lax.axis_index("s")
    wid = s * 2 + c
    lane = lax.iota(jnp.int32, 16)
    neginf = jnp.full((16,), -jnp.inf, jnp.float32)

    def initrow(r, _):
        for cc in range(8):
            acc[r, pl.ds(cc * 16, 16)] = neginf
        return 0

    lax.fori_loop(0, ACC_ROWS, initrow, 0)

    pltpu.sync_copy(counts_h.at[pl.ds(wid * NW, NW)], cntv)
    clo = cntv[pl.ds(0, 16)]
    chi = cntv[pl.ds(16, 16)]

    def src_body(v, _):
        vv = lane * 0 + v
        g_lo = clo.at[vv & 15].get(mode="promise_in_bounds")
        g_hi = chi.at[vv & 15].get(mode="promise_in_bounds")
        cntb = jnp.where(vv < 16, g_lo, g_hi)
        pkv[...] = cntb
        cnt_s = pkv[...][0]
        trips = (cnt_s + 15) >> 4
        lbase = wid * (NW * CAP) + v * CAP

        def grp(gg, _2):
            pltpu.sync_copy(lists_h.at[pl.ds(lbase + gg * 16, 16)], pkv)
            pk16 = pkv[...]
            valid = (lane + gg * 16) < cntb
            eidv = jnp.where(valid, pk16 & ((1 << EBITS) - 1), 0)
            rows = jnp.where(valid, pk16 >> EBITS, SAC)
            idxv[...] = eidv
            pltpu.async_copy(h3_h.at[idxv], rowbuf, semg).wait()
            for l in range(16):
                dd = rows[l]
                for cc in range(8):
                    sl = pl.ds(cc * 16, 16)
                    acc[dd, sl] = jnp.maximum(acc[dd, sl], rowbuf[l, sl])
            return 0

        lax.fori_loop(0, trips, grp, 0)
        return 0

    lax.fori_loop(0, NW, src_body, 0)

    def fixrow(r, _):
        for cc in range(8):
            sl = pl.ds(cc * 16, 16)
            vals = acc[r, sl]
            acc[r, sl] = jnp.where(jnp.isneginf(vals), 0.0, vals)
        return 0

    lax.fori_loop(0, RPW, fixrow, 0)

    @pl.when(wid < NW - 1)
    def _():
        pltpu.sync_copy(acc.at[pl.ds(0, RPW)],
                        out_h.at[pl.ds(wid * RPW, RPW)])

    @pl.when(wid == NW - 1)
    def _():
        last = N - (NW - 1) * RPW
        pltpu.sync_copy(acc.at[pl.ds(0, last)],
                        out_h.at[pl.ds((NW - 1) * RPW, last)])


def kernel(x, edge_index, W1, b1, W2, b2, W3, b3):
    w1a = W1[:D] - W1[D:]
    w1b = W1[D:]
    src = edge_index[0].astype(jnp.int32)
    dst = edge_index[1].astype(jnp.int32)

    a, b = pl.pallas_call(
        _node_mlp_kernel,
        grid=(5,),
        in_specs=[
            pl.BlockSpec((N // 5, D), lambda i: (i, 0)),
            pl.BlockSpec((D, D), lambda i: (0, 0)),
            pl.BlockSpec((D, D), lambda i: (0, 0)),
            pl.BlockSpec((1, D), lambda i: (0, 0)),
        ],
        out_specs=[
            pl.BlockSpec((N // 5, D), lambda i: (i, 0)),
            pl.BlockSpec((N // 5, D), lambda i: (i, 0)),
        ],
        out_shape=[
            jax.ShapeDtypeStruct((N, D), jnp.float32),
            jax.ShapeDtypeStruct((N, D), jnp.float32),
        ],
    )(x, w1a, w1b, b1.reshape(1, D))

    gather_bin = functools.partial(
        pl.kernel,
        out_type=[
            jax.ShapeDtypeStruct((E, D), jnp.float32),
            jax.ShapeDtypeStruct((NW * NW * CAP,), jnp.int32),
            jax.ShapeDtypeStruct((NW * NW,), jnp.int32),
        ],
        mesh=_mesh,
        scratch_types=[
            pltpu.VMEM((BATCH,), jnp.int32),
            pltpu.VMEM((BATCH,), jnp.int32),
            pltpu.VMEM((BATCH, D), jnp.float32),
            pltpu.VMEM((BATCH, D), jnp.float32),
            pltpu.VMEM((16,), jnp.int32),
            pltpu.VMEM((16,), jnp.int32),
            pltpu.SemaphoreType.DMA,
            pltpu.SemaphoreType.DMA,
            pltpu.SemaphoreType.DMA,
        ],
    )(_gather_bin_kernel)
    msg, lists, counts = gather_bin(a, b, src, dst)

    h3 = pl.pallas_call(
        _edge_mlp_kernel,
        grid=(160,),
        in_specs=[
            pl.BlockSpec((E // 160, D), lambda i: (i, 0)),
            pl.BlockSpec((D, D), lambda i: (0, 0)),
            pl.BlockSpec((1, D), lambda i: (0, 0)),
            pl.BlockSpec((D, D), lambda i: (0, 0)),
            pl.BlockSpec((1, D), lambda i: (0, 0)),
        ],
        out_specs=pl.BlockSpec((E // 160, D), lambda i: (i, 0)),
        out_shape=jax.ShapeDtypeStruct((E, D), jnp.float32),
    )(msg, W2, b2.reshape(1, D), W3, b3.reshape(1, D))

    segmax = functools.partial(
        pl.kernel,
        out_type=jax.ShapeDtypeStruct((N, D), jnp.float32),
        mesh=_mesh,
        scratch_types=[
            pltpu.VMEM((NW,), jnp.int32),
            pltpu.VMEM((16,), jnp.int32),
            pltpu.VMEM((16,), jnp.int32),
            pltpu.VMEM((16, D), jnp.float32),
            pltpu.VMEM((ACC_ROWS, D), jnp.float32),
            pltpu.SemaphoreType.DMA,
        ],
    )(_segmax_kernel)
    out = segmax(h3, lists, counts)
    return out


# speedup vs baseline: 2.4974x; 1.1433x over previous
import functools
import jax
import jax.numpy as jnp
from jax import lax
from jax.experimental import pallas as pl
from jax.experimental.pallas import tpu as pltpu
import jax.experimental.pallas.tpu_sc as plsc

N = 10000
E = 320000
D = 128
NW = 32
EPW = E // NW
RPW = 320
SAC = 320
ACC_ROWS = 328
CAP = EPW
MAGIC = 52429
EBITS = 19
BATCH = 80
NBATCH = EPW // BATCH
CHUNKW = 512

_mesh = plsc.VectorSubcoreMesh(core_axis_name="c", subcore_axis_name="s")


def _node_mlp_kernel(x_ref, w1a_ref, w1b_ref, b1_ref, a_ref, b_ref):
    x = x_ref[...]
    a_ref[...] = (
        jnp.dot(x, w1a_ref[...], preferred_element_type=jnp.float32)
        + b1_ref[...]
    )
    b_ref[...] = jnp.dot(x, w1b_ref[...], preferred_element_type=jnp.float32)


def _edge_mlp_kernel(msg_ref, w2_ref, b2_ref, w3_ref, b3_ref, out_ref):
    h = jnp.maximum(msg_ref[...], 0.0)
    h = jnp.maximum(
        jnp.dot(h, w2_ref[...], preferred_element_type=jnp.float32)
        + b2_ref[...],
        0.0,
    )
    out_ref[...] = (
        jnp.dot(h, w3_ref[...], preferred_element_type=jnp.float32)
        + b3_ref[...]
    )


def _gather_bin_kernel(a_h, b_h, src_h, dst_h, msg_h, lists_h, counts_h,
                       srcv, dstv, bufa, bufb, pkb, psb, sema, semb, semc):
    c = lax.axis_index("c")
    s = lax.axis_index("s")
    wid = s * 2 + c
    lane = lax.iota(jnp.int32, 16)
    ebase = wid * EPW

    def batch_body(t, carry):
        cnt_lo, cnt_hi = carry
        base = ebase + t * BATCH
        pltpu.sync_copy(dst_h.at[pl.ds(base, BATCH)], dstv)
        pltpu.sync_copy(src_h.at[pl.ds(base, BATCH)], srcv)
        cpa = pltpu.async_copy(a_h.at[dstv], bufa, sema)
        cpb = pltpu.async_copy(b_h.at[srcv], bufb, semb)

        for g in range(BATCH // 16):
            d16 = dstv[pl.ds(g * 16, 16)]
            bkt = (d16 * MAGIC) >> 24
            loc = d16 - bkt * RPW
            eid = (base + g * 16) + lane
            packed = loc * (1 << EBITS) + eid
            rank = jnp.zeros((16,), jnp.int32)
            hist_lo = jnp.zeros((16,), jnp.int32)
            hist_hi = jnp.zeros((16,), jnp.int32)
            for kk in range(16):
                skv = bkt.at[jnp.full((16,), kk, jnp.int32)].get(
                    mode="promise_in_bounds")
                eq = bkt == skv
                rank = rank + jnp.where(eq & (lane > kk), 1, 0)
                hist_lo = hist_lo + jnp.where(lane == skv, 1, 0)
                hist_hi = hist_hi + jnp.where((lane + 16) == skv, 1, 0)
            g_lo = cnt_lo.at[bkt & 15].get(mode="promise_in_bounds")
            g_hi = cnt_hi.at[bkt & 15].get(mode="promise_in_bounds")
            cnt16 = jnp.where(bkt < 16, g_lo, g_hi)
            pos = bkt * (NW * CAP) + wid * CAP + cnt16 + rank
            pkb[pl.ds(g * 16, 16)] = packed
            psb[pl.ds(g * 16, 16)] = pos
            cnt_lo = cnt_lo + hist_lo
            cnt_hi = cnt_hi + hist_hi
        cps = pltpu.async_copy(pkb, lists_h.at[psb], semc)

        cpa.wait()
        cpb.wait()

        def row_body(r, _):
            for cc in range(8):
                sl = pl.ds(cc * 16, 16)
                bufa[r, sl] = bufa[r, sl] + bufb[r, sl]
            return 0

        lax.fori_loop(0, BATCH, row_body, 0)
        pltpu.sync_copy(bufa, msg_h.at[pl.ds(base, BATCH)])
        cps.wait()
        return cnt_lo, cnt_hi

    z16 = jnp.zeros((16,), jnp.int32)
    cnt_lo, cnt_hi = lax.fori_loop(0, NBATCH, batch_body, (z16, z16))

    pkb[pl.ds(0, 16)] = cnt_lo
    psb[pl.ds(0, 16)] = lane * NW + wid
    pkb[pl.ds(16, 16)] = cnt_hi
    psb[pl.ds(16, 16)] = (lane + 16) * NW + wid
    for g in range(2, BATCH // 16):
        psb[pl.ds(g * 16, 16)] = NW * NW + (g - 2) * 16 + lane
    pltpu.async_copy(pkb, counts_h.at[psb], semc).wait()


def _segmax_kernel(h3_h, lists_h, counts_h, out_h,
                   cntv, pkbuf, idxv2, rowbuf2, acc, semg):
    c = lax.axis_index("c")
    s = lax.axis_index("s")
    wid = s * 2 + c
    lane = lax.iota(jnp.int32, 16)
    neginf = jnp.full((16,), -jnp.inf, jnp.float32)

    def initrow(r, _):
        for cc in range(8):
            acc[r, pl.ds(cc * 16, 16)] = neginf
        return 0

    lax.fori_loop(0, ACC_ROWS, initrow, 0)

    pltpu.sync_copy(counts_h.at[pl.ds(wid * NW, NW)], cntv)
    clo = cntv[pl.ds(0, 16)]
    chi = cntv[pl.ds(16, 16)]

    def src_body(v, _):
        vv = lane * 0 + v
        g_lo = clo.at[vv & 15].get(mode="promise_in_bounds")
        g_hi = chi.at[vv & 15].get(mode="promise_in_bounds")
        cntb = jnp.where(vv < 16, g_lo, g_hi)
        idxv2[pl.ds(0, 16)] = cntb
        cnt_s = idxv2[pl.ds(0, 16)][0]
        trips = (cnt_s + 15) >> 4
        nchunks = (cnt_s + (CHUNKW - 1)) >> 9
        lbase = wid * (NW * CAP) + v * CAP

        def prep(j_local, g_abs, slot):
            pk16 = pkbuf[pl.ds(j_local * 16, 16)]
            valid = (lane + g_abs * 16) < cntb
            eidv = jnp.where(valid, pk16 & ((1 << EBITS) - 1), 0)
            rows = jnp.where(valid, pk16 >> EBITS, SAC)
            idxv2[pl.ds(slot * 16, 16)] = eidv
            pltpu.async_copy(h3_h.at[idxv2.at[pl.ds(slot * 16, 16)]],
                             rowbuf2.at[pl.ds(slot * 16, 16)],
                             semg.at[slot])
            return rows

        def chunk_body(cidx, _2):
            pltpu.sync_copy(
                lists_h.at[pl.ds(lbase + cidx * CHUNKW, CHUNKW)], pkbuf)
            gbase = cidx * (CHUNKW // 16)
            ngrp = jnp.minimum(trips - gbase, CHUNKW // 16)
            rows0 = prep(0, gbase, 0)

            def grp(gg, rows_cur):
                slot = gg & 1
                nj = jnp.minimum(gg + 1, ngrp - 1)
                rows_nxt = prep(nj, gbase + nj, (gg + 1) & 1)
                pltpu.make_async_copy(
                    h3_h.at[idxv2.at[pl.ds(slot * 16, 16)]],
                    rowbuf2.at[pl.ds(slot * 16, 16)], semg.at[slot]).wait()
                for l in range(16):
                    dd = rows_cur[l]
                    for cc in range(8):
                        sl = pl.ds(cc * 16, 16)
                        acc[dd, sl] = jnp.maximum(acc[dd, sl],
                                                  rowbuf2[slot * 16 + l, sl])
                return rows_nxt

            lax.fori_loop(0, ngrp, grp, rows0)
            lastslot = ngrp & 1
            pltpu.make_async_copy(
                h3_h.at[idxv2.at[pl.ds(lastslot * 16, 16)]],
                rowbuf2.at[pl.ds(lastslot * 16, 16)],
                semg.at[lastslot]).wait()
            return 0

        lax.fori_loop(0, nchunks, chunk_body, 0)
        return 0

    lax.fori_loop(0, NW, src_body, 0)

    def fixrow(r, _):
        for cc in range(8):
            sl = pl.ds(cc * 16, 16)
            vals = acc[r, sl]
            acc[r, sl] = jnp.where(jnp.isneginf(vals), 0.0, vals)
        return 0

    lax.fori_loop(0, RPW, fixrow, 0)

    @pl.when(wid < NW - 1)
    def _():
        pltpu.sync_copy(acc.at[pl.ds(0, RPW)],
                        out_h.at[pl.ds(wid * RPW, RPW)])

    @pl.when(wid == NW - 1)
    def _():
        last = N - (NW - 1) * RPW
        pltpu.sync_copy(acc.at[pl.ds(0, last)],
                        out_h.at[pl.ds((NW - 1) * RPW, last)])


def kernel(x, edge_index, W1, b1, W2, b2, W3, b3):
    w1a = W1[:D] - W1[D:]
    w1b = W1[D:]
    src = edge_index[0].astype(jnp.int32)
    dst = edge_index[1].astype(jnp.int32)

    a, b = pl.pallas_call(
        _node_mlp_kernel,
        grid=(5,),
        in_specs=[
            pl.BlockSpec((N // 5, D), lambda i: (i, 0)),
            pl.BlockSpec((D, D), lambda i: (0, 0)),
            pl.BlockSpec((D, D), lambda i: (0, 0)),
            pl.BlockSpec((1, D), lambda i: (0, 0)),
        ],
        out_specs=[
            pl.BlockSpec((N // 5, D), lambda i: (i, 0)),
            pl.BlockSpec((N // 5, D), lambda i: (i, 0)),
        ],
        out_shape=[
            jax.ShapeDtypeStruct((N, D), jnp.float32),
            jax.ShapeDtypeStruct((N, D), jnp.float32),
        ],
    )(x, w1a, w1b, b1.reshape(1, D))

    gather_bin = functools.partial(
        pl.kernel,
        out_type=[
            jax.ShapeDtypeStruct((E, D), jnp.float32),
            jax.ShapeDtypeStruct((NW * NW * CAP + CHUNKW,), jnp.int32),
            jax.ShapeDtypeStruct((NW * NW + BATCH - 32,), jnp.int32),
        ],
        mesh=_mesh,
        scratch_types=[
            pltpu.VMEM((BATCH,), jnp.int32),
            pltpu.VMEM((BATCH,), jnp.int32),
            pltpu.VMEM((BATCH, D), jnp.float32),
            pltpu.VMEM((BATCH, D), jnp.float32),
            pltpu.VMEM((BATCH,), jnp.int32),
            pltpu.VMEM((BATCH,), jnp.int32),
            pltpu.SemaphoreType.DMA,
            pltpu.SemaphoreType.DMA,
            pltpu.SemaphoreType.DMA,
        ],
    )(_gather_bin_kernel)
    msg, lists, counts = gather_bin(a, b, src, dst)

    h3 = pl.pallas_call(
        _edge_mlp_kernel,
        grid=(160,),
        in_specs=[
            pl.BlockSpec((E // 160, D), lambda i: (i, 0)),
            pl.BlockSpec((D, D), lambda i: (0, 0)),
            pl.BlockSpec((1, D), lambda i: (0, 0)),
            pl.BlockSpec((D, D), lambda i: (0, 0)),
            pl.BlockSpec((1, D), lambda i: (0, 0)),
        ],
        out_specs=pl.BlockSpec((E // 160, D), lambda i: (i, 0)),
        out_shape=jax.ShapeDtypeStruct((E, D), jnp.float32),
    )(msg, W2, b2.reshape(1, D), W3, b3.reshape(1, D))

    segmax = functools.partial(
        pl.kernel,
        out_type=jax.ShapeDtypeStruct((N, D), jnp.float32),
        mesh=_mesh,
        scratch_types=[
            pltpu.VMEM((NW,), jnp.int32),
            pltpu.VMEM((CHUNKW,), jnp.int32),
            pltpu.VMEM((32,), jnp.int32),
            pltpu.VMEM((32, D), jnp.float32),
            pltpu.VMEM((ACC_ROWS, D), jnp.float32),
            pltpu.SemaphoreType.DMA((2,)),
        ],
    )(_segmax_kernel)
    out = segmax(h3, lists, counts)
    return out


# speedup vs baseline: 2.5332x; 1.0143x over previous
import functools
import jax
import jax.numpy as jnp
from jax import lax
from jax.experimental import pallas as pl
from jax.experimental.pallas import tpu as pltpu
import jax.experimental.pallas.tpu_sc as plsc

N = 10000
E = 320000
D = 128
NW = 32
EPW = E // NW
RPW = 320
SAC = 320
ACC_ROWS = 328
CAP = EPW
MAGIC = 52429
EBITS = 19
BATCH = 80
NBATCH = EPW // BATCH
CHUNKW = 512

_mesh = plsc.VectorSubcoreMesh(core_axis_name="c", subcore_axis_name="s")


def _node_mlp_kernel(x_ref, w1a_ref, w1b_ref, b1_ref, a_ref, b_ref):
    x = x_ref[...]
    a_ref[...] = (
        jnp.dot(x, w1a_ref[...], preferred_element_type=jnp.float32)
        + b1_ref[...]
    )
    b_ref[...] = jnp.dot(x, w1b_ref[...], preferred_element_type=jnp.float32)


def _edge_mlp_kernel(msg_ref, w2_ref, b2_ref, w3_ref, b3_ref, out_ref):
    h = jnp.maximum(msg_ref[...], 0.0)
    h = jnp.maximum(
        jnp.dot(h, w2_ref[...], preferred_element_type=jnp.float32)
        + b2_ref[...],
        0.0,
    )
    out_ref[...] = (
        jnp.dot(h, w3_ref[...], preferred_element_type=jnp.float32)
        + b3_ref[...]
    )


def _gather_bin_kernel(a_h, b_h, src_h, dst_h, msg_h, lists_h, counts_h,
                       srcv, dstv, bufa, bufb, pkb, psb,
                       sema, semb, semc, semm):
    c = lax.axis_index("c")
    s = lax.axis_index("s")
    wid = s * 2 + c
    lane = lax.iota(jnp.int32, 16)
    ebase = wid * EPW

    def emit_batch(t, s_, drain, counts):
        cnt_lo, cnt_hi = counts
        base = ebase + t * BATCH
        off = s_ * BATCH
        pkh = pkb.at[s_]
        psh = psb.at[s_]
        bah = bufa.at[pl.ds(off, BATCH)]
        if drain:
            pltpu.make_async_copy(
                bah, msg_h.at[pl.ds(base, BATCH)], semm.at[s_]).wait()
            pltpu.make_async_copy(
                pkh, lists_h.at[psh], semc.at[s_]).wait()
        pltpu.sync_copy(dst_h.at[pl.ds(base, BATCH)], dstv)
        pltpu.sync_copy(src_h.at[pl.ds(base, BATCH)], srcv)
        cpa = pltpu.async_copy(a_h.at[dstv], bah, sema)
        cpb = pltpu.async_copy(b_h.at[srcv], bufb, semb)

        for g in range(BATCH // 16):
            d16 = dstv[pl.ds(g * 16, 16)]
            bkt = (d16 * MAGIC) >> 24
            loc = d16 - bkt * RPW
            eid = (base + g * 16) + lane
            packed = loc * (1 << EBITS) + eid
            rank = jnp.zeros((16,), jnp.int32)
            hist_lo = jnp.zeros((16,), jnp.int32)
            hist_hi = jnp.zeros((16,), jnp.int32)
            for kk in range(16):
                skv = bkt.at[jnp.full((16,), kk, jnp.int32)].get(
                    mode="promise_in_bounds")
                eq = bkt == skv
                rank = rank + jnp.where(eq & (lane > kk), 1, 0)
                hist_lo = hist_lo + jnp.where(lane == skv, 1, 0)
                hist_hi = hist_hi + jnp.where((lane + 16) == skv, 1, 0)
            g_lo = cnt_lo.at[bkt & 15].get(mode="promise_in_bounds")
            g_hi = cnt_hi.at[bkt & 15].get(mode="promise_in_bounds")
            cnt16 = jnp.where(bkt < 16, g_lo, g_hi)
            pos = bkt * (NW * CAP) + wid * CAP + cnt16 + rank
            pkb[s_, pl.ds(g * 16, 16)] = packed
            psb[s_, pl.ds(g * 16, 16)] = pos
            cnt_lo = cnt_lo + hist_lo
            cnt_hi = cnt_hi + hist_hi
        pltpu.async_copy(pkh, lists_h.at[psh], semc.at[s_])

        cpa.wait()
        cpb.wait()

        def row_body(r, _):
            for cc in range(8):
                sl = pl.ds(cc * 16, 16)
                bufa[off + r, sl] = bufa[off + r, sl] + bufb[r, sl]
            return 0

        lax.fori_loop(0, BATCH, row_body, 0)
        pltpu.async_copy(bah, msg_h.at[pl.ds(base, BATCH)], semm.at[s_])
        return cnt_lo, cnt_hi

    z16 = jnp.zeros((16,), jnp.int32)
    counts = (z16, z16)
    counts = emit_batch(0, 0, False, counts)
    counts = emit_batch(1, 1, False, counts)

    def pair_body(t2, counts):
        counts = emit_batch(2 * t2, 0, True, counts)
        counts = emit_batch(2 * t2 + 1, 1, True, counts)
        return counts

    counts = lax.fori_loop(1, (NBATCH - 1) // 2, pair_body, counts)
    counts = emit_batch(NBATCH - 1, 0, True, counts)
    cnt_lo, cnt_hi = counts

    pltpu.make_async_copy(bufa.at[pl.ds(0, BATCH)],
                          msg_h.at[pl.ds(0, BATCH)], semm.at[0]).wait()
    pltpu.make_async_copy(pkb.at[0], lists_h.at[psb.at[0]],
                          semc.at[0]).wait()
    pltpu.make_async_copy(bufa.at[pl.ds(BATCH, BATCH)],
                          msg_h.at[pl.ds(0, BATCH)], semm.at[1]).wait()
    pltpu.make_async_copy(pkb.at[1], lists_h.at[psb.at[1]],
                          semc.at[1]).wait()

    pkb[0, pl.ds(0, 16)] = cnt_lo
    psb[0, pl.ds(0, 16)] = lane * NW + wid
    pkb[0, pl.ds(16, 16)] = cnt_hi
    psb[0, pl.ds(16, 16)] = (lane + 16) * NW + wid
    for g in range(2, BATCH // 16):
        psb[0, pl.ds(g * 16, 16)] = NW * NW + (g - 2) * 16 + lane
    pltpu.async_copy(pkb.at[0], counts_h.at[psb.at[0]], semc.at[0]).wait()


def _segmax_kernel(h3_h, lists_h, counts_h, out_h,
                   cntv, pkbuf, idxv2, rowbuf2, acc, semg):
    c = lax.axis_index("c")
    s = lax.axis_index("s")
    wid = s * 2 + c
    lane = lax.iota(jnp.int32, 16)
    neginf = jnp.full((16,), -jnp.inf, jnp.float32)

    def initrow(r, _):
        for cc in range(8):
            acc[r, pl.ds(cc * 16, 16)] = neginf
        return 0

    lax.fori_loop(0, ACC_ROWS, initrow, 0)

    pltpu.sync_copy(counts_h.at[pl.ds(wid * NW, NW)], cntv)
    clo = cntv[pl.ds(0, 16)]
    chi = cntv[pl.ds(16, 16)]

    def src_body(v, _):
        vv = lane * 0 + v
        g_lo = clo.at[vv & 15].get(mode="promise_in_bounds")
        g_hi = chi.at[vv & 15].get(mode="promise_in_bounds")
        cntb = jnp.where(vv < 16, g_lo, g_hi)
        idxv2[pl.ds(0, 16)] = cntb
        cnt_s = idxv2[pl.ds(0, 16)][0]
        trips = (cnt_s + 15) >> 4
        nchunks = (cnt_s + (CHUNKW - 1)) >> 9
        lbase = wid * (NW * CAP) + v * CAP

        def prep(j_local, g_abs, slot):
            pk16 = pkbuf[pl.ds(j_local * 16, 16)]
            valid = (lane + g_abs * 16) < cntb
            eidv = jnp.where(valid, pk16 & ((1 << EBITS) - 1), 0)
            rows = jnp.where(valid, pk16 >> EBITS, SAC)
            idxv2[pl.ds(slot * 16, 16)] = eidv
            pltpu.async_copy(h3_h.at[idxv2.at[pl.ds(slot * 16, 16)]],
                             rowbuf2.at[pl.ds(slot * 16, 16)],
                             semg.at[slot])
            return rows

        def chunk_body(cidx, _2):
            pltpu.sync_copy(
                lists_h.at[pl.ds(lbase + cidx * CHUNKW, CHUNKW)], pkbuf)
            gbase = cidx * (CHUNKW // 16)
            ngrp = jnp.minimum(trips - gbase, CHUNKW // 16)
            rows0 = prep(0, gbase, 0)

            def grp(gg, rows_cur):
                slot = gg & 1
                nj = jnp.minimum(gg + 1, ngrp - 1)
                rows_nxt = prep(nj, gbase + nj, (gg + 1) & 1)
                pltpu.make_async_copy(
                    h3_h.at[idxv2.at[pl.ds(slot * 16, 16)]],
                    rowbuf2.at[pl.ds(slot * 16, 16)], semg.at[slot]).wait()
                for l in range(16):
                    dd = rows_cur[l]
                    for cc in range(8):
                        sl = pl.ds(cc * 16, 16)
                        acc[dd, sl] = jnp.maximum(acc[dd, sl],
                                                  rowbuf2[slot * 16 + l, sl])
                return rows_nxt

            lax.fori_loop(0, ngrp, grp, rows0)
            lastslot = ngrp & 1
            pltpu.make_async_copy(
                h3_h.at[idxv2.at[pl.ds(lastslot * 16, 16)]],
                rowbuf2.at[pl.ds(lastslot * 16, 16)],
                semg.at[lastslot]).wait()
            return 0

        lax.fori_loop(0, nchunks, chunk_body, 0)
        return 0

    lax.fori_loop(0, NW, src_body, 0)

    def fixrow(r, _):
        for cc in range(8):
            sl = pl.ds(cc * 16, 16)
            vals = acc[r, sl]
            acc[r, sl] = jnp.where(jnp.isneginf(vals), 0.0, vals)
        return 0

    lax.fori_loop(0, RPW, fixrow, 0)

    @pl.when(wid < NW - 1)
    def _():
        pltpu.sync_copy(acc.at[pl.ds(0, RPW)],
                        out_h.at[pl.ds(wid * RPW, RPW)])

    @pl.when(wid == NW - 1)
    def _():
        last = N - (NW - 1) * RPW
        pltpu.sync_copy(acc.at[pl.ds(0, last)],
                        out_h.at[pl.ds((NW - 1) * RPW, last)])


def kernel(x, edge_index, W1, b1, W2, b2, W3, b3):
    w1a = W1[:D] - W1[D:]
    w1b = W1[D:]
    src = edge_index[0].astype(jnp.int32)
    dst = edge_index[1].astype(jnp.int32)

    a, b = pl.pallas_call(
        _node_mlp_kernel,
        grid=(5,),
        in_specs=[
            pl.BlockSpec((N // 5, D), lambda i: (i, 0)),
            pl.BlockSpec((D, D), lambda i: (0, 0)),
            pl.BlockSpec((D, D), lambda i: (0, 0)),
            pl.BlockSpec((1, D), lambda i: (0, 0)),
        ],
        out_specs=[
            pl.BlockSpec((N // 5, D), lambda i: (i, 0)),
            pl.BlockSpec((N // 5, D), lambda i: (i, 0)),
        ],
        out_shape=[
            jax.ShapeDtypeStruct((N, D), jnp.float32),
            jax.ShapeDtypeStruct((N, D), jnp.float32),
        ],
    )(x, w1a, w1b, b1.reshape(1, D))

    gather_bin = functools.partial(
        pl.kernel,
        out_type=[
            jax.ShapeDtypeStruct((E, D), jnp.float32),
            jax.ShapeDtypeStruct((NW * NW * CAP + CHUNKW,), jnp.int32),
            jax.ShapeDtypeStruct((NW * NW + BATCH - 32,), jnp.int32),
        ],
        mesh=_mesh,
        scratch_types=[
            pltpu.VMEM((BATCH,), jnp.int32),
            pltpu.VMEM((BATCH,), jnp.int32),
            pltpu.VMEM((2 * BATCH, D), jnp.float32),
            pltpu.VMEM((BATCH, D), jnp.float32),
            pltpu.VMEM((2, BATCH), jnp.int32),
            pltpu.VMEM((2, BATCH), jnp.int32),
            pltpu.SemaphoreType.DMA,
            pltpu.SemaphoreType.DMA,
            pltpu.SemaphoreType.DMA((2,)),
            pltpu.SemaphoreType.DMA((2,)),
        ],
    )(_gather_bin_kernel)
    msg, lists, counts = gather_bin(a, b, src, dst)

    h3 = pl.pallas_call(
        _edge_mlp_kernel,
        grid=(160,),
        in_specs=[
            pl.BlockSpec((E // 160, D), lambda i: (i, 0)),
            pl.BlockSpec((D, D), lambda i: (0, 0)),
            pl.BlockSpec((1, D), lambda i: (0, 0)),
            pl.BlockSpec((D, D), lambda i: (0, 0)),
            pl.BlockSpec((1, D), lambda i: (0, 0)),
        ],
        out_specs=pl.BlockSpec((E // 160, D), lambda i: (i, 0)),
        out_shape=jax.ShapeDtypeStruct((E, D), jnp.float32),
    )(msg, W2, b2.reshape(1, D), W3, b3.reshape(1, D))

    segmax = functools.partial(
        pl.kernel,
        out_type=jax.ShapeDtypeStruct((N, D), jnp.float32),
        mesh=_mesh,
        scratch_types=[
            pltpu.VMEM((NW,), jnp.int32),
            pltpu.VMEM((CHUNKW,), jnp.int32),
            pltpu.VMEM((32,), jnp.int32),
            pltpu.VMEM((32, D), jnp.float32),
            pltpu.VMEM((ACC_ROWS, D), jnp.float32),
            pltpu.SemaphoreType.DMA((2,)),
        ],
    )(_segmax_kernel)
    out = segmax(h3, lists, counts)
    return out
